# baseline (speedup 1.0000x reference)
import functools

import jax
import jax.numpy as jnp
from jax import lax
from jax.experimental import pallas as pl
from jax.experimental.pallas import tpu as pltpu
from jax.experimental.pallas import tpu_sc as plsc

NC, NS, LANES = 2, 16, 16
NW = NC * NS

_MESH = dict(core_axis_name="c", subcore_axis_name="s", num_cores=NC,
             num_subcores=NS)
_SC_PARAMS = pltpu.CompilerParams(needs_layout_passes=False)


_K = 80
_CH = 80


def _make_agg(N, C, E):
    K = _K
    ept = E // NW
    nb = ept // K
    nt = nb // 2
    nch = N // _CH
    tmax = -(-nch // NS)
    mesh = plsc.VectorSubcoreMesh(**_MESH)

    @functools.partial(
        pl.kernel,
        out_type=jax.ShapeDtypeStruct((NC, N, C), jnp.float32),
        mesh=mesh,
        scratch_types=[
            pltpu.VMEM((K,), jnp.int32),
            pltpu.VMEM((K,), jnp.int32),
            pltpu.VMEM((K,), jnp.int32),
            pltpu.VMEM((K,), jnp.int32),
            pltpu.VMEM((K,), jnp.int32),
            pltpu.VMEM((K,), jnp.int32),
            pltpu.VMEM((K,), jnp.float32),
            pltpu.VMEM((K,), jnp.float32),
            pltpu.VMEM((K, C), jnp.float32),
            pltpu.VMEM((K, C), jnp.float32),
            pltpu.VMEM_SHARED((N, C), jnp.float32),
            pltpu.SemaphoreType.DMA,
            pltpu.SemaphoreType.DMA,
            pltpu.SemaphoreType.DMA,
            pltpu.SemaphoreType.DMA,
        ],
        compiler_params=_SC_PARAMS,
    )
    def agg(g_hbm, src_hbm, dst_hbm, w_hbm, out_hbm,
            src_a, src_b, dst_a, dst_b, dsts_a, dsts_b, w_a, w_b,
            rows_a, rows_b, acc_sh,
            sem_ia, sem_ib, sem_ga, sem_gb):
        c = lax.axis_index("c")
        s = lax.axis_index("s")
        wid = c * NS + s
        base = wid * ept

        zv = jnp.zeros((LANES,), jnp.float32)

        def zrow(r, carry):
            for j in range(C // LANES):
                rows_a[r, pl.ds(j * LANES, LANES)] = zv
            return carry

        lax.fori_loop(0, _CH, zrow, 0)
        for t in range(tmax):
            ck = s + NS * t

            @pl.when(ck < nch)
            def _():
                pltpu.sync_copy(rows_a.at[pl.ds(0, _CH)],
                                acc_sh.at[pl.ds(ck * _CH, _CH)])

        plsc.subcore_barrier()

        def idx_descs(i, srcb, dstb, wb, sem):
            off = base + i * K
            return (
                pltpu.make_async_copy(src_hbm.at[pl.ds(off, K)], srcb, sem),
                pltpu.make_async_copy(dst_hbm.at[pl.ds(off, K)], dstb, sem),
                pltpu.make_async_copy(w_hbm.at[pl.ds(off, K)], wb, sem),
            )

        def idx_load(i, srcb, dstb, wb, sem):
            for d in idx_descs(i, srcb, dstb, wb, sem):
                d.start()

        def idx_wait(i, srcb, dstb, wb, sem):
            for d in idx_descs(i, srcb, dstb, wb, sem):
                d.wait()

        def gath(rows, srcb, sem):
            return pltpu.make_async_copy(g_hbm.at[srcb], rows, sem)

        def scale(rows, wb):
            def srow(r, carry):
                for rr in (2 * r, 2 * r + 1):
                    wv = plsc.load_gather(
                        wb, [jnp.full((LANES,), rr, jnp.int32)])
                    for j in range(C // LANES):
                        sl = pl.ds(j * LANES, LANES)
                        rows[rr, sl] = rows[rr, sl] * wv
                return carry

            lax.fori_loop(0, K // 2, srow, 0)

        def cpidx(dstb, dstsb):
            for j in range(K // LANES):
                sl = pl.ds(j * LANES, LANES)
                dstsb[sl] = dstb[sl]

        idx_load(0, src_a, dst_a, w_a, sem_ia)
        idx_load(1, src_b, dst_b, w_b, sem_ib)
        idx_wait(0, src_a, dst_a, w_a, sem_ia)

        def step(t, carry):
            i0 = 2 * t
            i1 = i0 + 1

            idx_wait(i1, src_b, dst_b, w_b, sem_ib)
            gath(rows_a, src_a, sem_ga).wait()
            scale(rows_a, w_a)
            cpidx(dst_a, dsts_a)
            gath(rows_b, src_b, sem_gb).start()
            pltpu.sync_copy(rows_a, acc_sh.at[dsts_a], add=True)

            @pl.when(i0 + 2 < nb)
            def _():
                idx_load(i0 + 2, src_a, dst_a, w_a, sem_ia)
                idx_wait(i0 + 2, src_a, dst_a, w_a, sem_ia)

            gath(rows_b, src_b, sem_gb).wait()
            scale(rows_b, w_b)
            cpidx(dst_b, dsts_b)

            @pl.when(i0 + 2 < nb)
            def _():
                gath(rows_a, src_a, sem_ga).start()

            pltpu.sync_copy(rows_b, acc_sh.at[dsts_b], add=True)

            @pl.when(i1 + 2 < nb)
            def _():
                idx_load(i1 + 2, src_b, dst_b, w_b, sem_ib)

            return carry

        gath(rows_a, src_a, sem_ga).start()
        lax.fori_loop(0, nt, step, 0)

        if nb % 2:
            last = nb - 1
            gath(rows_a, src_a, sem_ga).wait()
            scale(rows_a, w_a)
            cpidx(dst_a, dsts_a)
            pltpu.sync_copy(rows_a, acc_sh.at[dsts_a], add=True)

        plsc.subcore_barrier()
        for t in range(tmax):
            ck = s + NS * t

            @pl.when(ck < nch)
            def _():
                pltpu.sync_copy(acc_sh.at[pl.ds(ck * _CH, _CH)],
                                out_hbm.at[c, pl.ds(ck * _CH, _CH)])

    return agg


def _make_deg(N, E):
    C = LANES
    K = _K
    ept = E // NW
    nb = ept // K
    nt = nb // 2
    nch = N // _CH
    tmax = -(-nch // NS)
    mesh = plsc.VectorSubcoreMesh(**_MESH)

    @functools.partial(
        pl.kernel,
        out_type=jax.ShapeDtypeStruct((NC, N, C), jnp.float32),
        mesh=mesh,
        scratch_types=[
            pltpu.VMEM((K,), jnp.int32),
            pltpu.VMEM((K,), jnp.int32),
            pltpu.VMEM((K,), jnp.int32),
            pltpu.VMEM((K,), jnp.int32),
            pltpu.VMEM((K,), jnp.float32),
            pltpu.VMEM((K,), jnp.float32),
            pltpu.VMEM((K, C), jnp.float32),
            pltpu.VMEM((K, C), jnp.float32),
            pltpu.VMEM_SHARED((N, C), jnp.float32),
            pltpu.SemaphoreType.DMA,
            pltpu.SemaphoreType.DMA,
        ],
        compiler_params=_SC_PARAMS,
    )
    def deg(dst_hbm, w_hbm, out_hbm,
            dst_a, dst_b, dsts_a, dsts_b, w_a, w_b, rows_a, rows_b,
            acc_sh, sem_ia, sem_ib):
        c = lax.axis_index("c")
        s = lax.axis_index("s")
        wid = c * NS + s
        base = wid * ept

        zv = jnp.zeros((LANES,), jnp.float32)

        def zrow(r, carry):
            rows_a[r, pl.ds(0, LANES)] = zv
            return carry

        lax.fori_loop(0, _CH, zrow, 0)
        for t in range(tmax):
            ck = s + NS * t

            @pl.when(ck < nch)
            def _():
                pltpu.sync_copy(rows_a.at[pl.ds(0, _CH)],
                                acc_sh.at[pl.ds(ck * _CH, _CH)])

        plsc.subcore_barrier()

        def idx_descs(i, dstb, wb, sem):
            off = base + i * K
            return (
                pltpu.make_async_copy(dst_hbm.at[pl.ds(off, K)], dstb, sem),
                pltpu.make_async_copy(w_hbm.at[pl.ds(off, K)], wb, sem),
            )

        def idx_load(i, dstb, wb, sem):
            for d in idx_descs(i, dstb, wb, sem):
                d.start()

        def idx_wait(i, dstb, wb, sem):
            for d in idx_descs(i, dstb, wb, sem):
                d.wait()

        def fill(rows, wb):
            def frow(r, carry):
                for rr in (2 * r, 2 * r + 1):
                    wv = plsc.load_gather(
                        wb, [jnp.full((LANES,), rr, jnp.int32)])
                    rows[rr, pl.ds(0, LANES)] = wv
                return carry

            lax.fori_loop(0, K // 2, frow, 0)

        def cpidx(dstb, dstsb):
            for j in range(K // LANES):
                sl = pl.ds(j * LANES, LANES)
                dstsb[sl] = dstb[sl]

        idx_load(0, dst_a, w_a, sem_ia)
        idx_load(1, dst_b, w_b, sem_ib)
        idx_wait(0, dst_a, w_a, sem_ia)

        def step(t, carry):
            i0 = 2 * t
            i1 = i0 + 1

            idx_wait(i1, dst_b, w_b, sem_ib)
            fill(rows_a, w_a)
            cpidx(dst_a, dsts_a)
            pltpu.sync_copy(rows_a, acc_sh.at[dsts_a], add=True)

            @pl.when(i0 + 2 < nb)
            def _():
                idx_load(i0 + 2, dst_a, w_a, sem_ia)
                idx_wait(i0 + 2, dst_a, w_a, sem_ia)

            fill(rows_b, w_b)
            cpidx(dst_b, dsts_b)
            pltpu.sync_copy(rows_b, acc_sh.at[dsts_b], add=True)

            @pl.when(i1 + 2 < nb)
            def _():
                idx_load(i1 + 2, dst_b, w_b, sem_ib)

            return carry

        lax.fori_loop(0, nt, step, 0)

        if nb % 2:
            last = nb - 1
            fill(rows_a, w_a)
            cpidx(dst_a, dsts_a)
            pltpu.sync_copy(rows_a, acc_sh.at[dsts_a], add=True)

        plsc.subcore_barrier()
        for t in range(tmax):
            ck = s + NS * t

            @pl.when(ck < nch)
            def _():
                pltpu.sync_copy(acc_sh.at[pl.ds(ck * _CH, _CH)],
                                out_hbm.at[c, pl.ds(ck * _CH, _CH)])

    return deg


_BR = 400


def _dis_of(degp_ref):
    deg = degp_ref[0] + degp_ref[1]
    return lax.rsqrt(deg[:, 0:1] + 1.0)


def _mm_scale(x, W, degp):
    N, IC = x.shape
    H = W.shape[1]

    def body(x_ref, w_ref, degp_ref, o_ref):
        dis = _dis_of(degp_ref)
        h = jnp.dot(x_ref[...], w_ref[...], preferred_element_type=jnp.float32)
        o_ref[...] = h * dis

    return pl.pallas_call(
        body,
        grid=(N // _BR,),
        in_specs=[
            pl.BlockSpec((_BR, IC), lambda i: (i, 0)),
            pl.BlockSpec((IC, H), lambda i: (0, 0)),
            pl.BlockSpec((NC, _BR, LANES), lambda i: (0, i, 0)),
        ],
        out_specs=pl.BlockSpec((_BR, H), lambda i: (i, 0)),
        out_shape=jax.ShapeDtypeStruct((N, H), jnp.float32),
    )(x, W, degp)


def _mid(p, g1, degp, b1, W2):
    N, H = g1.shape
    O = W2.shape[1]

    def body(p_ref, g1_ref, degp_ref, b1_ref, w2_ref, o_ref):
        dis = _dis_of(degp_ref)
        t = dis * (p_ref[0] + p_ref[1] + g1_ref[...]) + b1_ref[...]
        h1 = jnp.maximum(t, 0.0)
        h2 = jnp.dot(h1, w2_ref[...], preferred_element_type=jnp.float32)
        o_ref[...] = h2 * dis

    return pl.pallas_call(
        body,
        grid=(N // _BR,),
        in_specs=[
            pl.BlockSpec((NC, _BR, H), lambda i: (0, i, 0)),
            pl.BlockSpec((_BR, H), lambda i: (i, 0)),
            pl.BlockSpec((NC, _BR, LANES), lambda i: (0, i, 0)),
            pl.BlockSpec((1, H), lambda i: (0, 0)),
            pl.BlockSpec((H, O), lambda i: (0, 0)),
        ],
        out_specs=pl.BlockSpec((_BR, O), lambda i: (i, 0)),
        out_shape=jax.ShapeDtypeStruct((N, O), jnp.float32),
    )(p, g1, degp, b1, W2)


def _final(p, g2, degp, b2, O):
    N, OP = g2.shape

    def body(p_ref, g2_ref, degp_ref, b2_ref, o_ref):
        dis = _dis_of(degp_ref)
        o = (dis * (p_ref[0] + p_ref[1] + g2_ref[...]) + b2_ref[...])[:, :O]
        m = jnp.max(o, axis=1, keepdims=True)
        e = jnp.exp(o - m)
        lse = jnp.log(jnp.sum(e, axis=1, keepdims=True))
        o_ref[...] = o - m - lse

    return pl.pallas_call(
        body,
        grid=(N // _BR,),
        in_specs=[
            pl.BlockSpec((NC, _BR, OP), lambda i: (0, i, 0)),
            pl.BlockSpec((_BR, OP), lambda i: (i, 0)),
            pl.BlockSpec((NC, _BR, LANES), lambda i: (0, i, 0)),
            pl.BlockSpec((1, OP), lambda i: (0, 0)),
        ],
        out_specs=pl.BlockSpec((_BR, O), lambda i: (i, 0)),
        out_shape=jax.ShapeDtypeStruct((N, O), jnp.float32),
    )(p, g2, degp, b2)


def kernel(x, adj, edge_weights, W1, b1, W2, b2):
    N, IC = x.shape
    H = W1.shape[1]
    O = W2.shape[1]
    E = adj.shape[1]
    src = adj[0]
    dst = adj[1]

    OP = 128
    W2p = jnp.pad(W2, ((0, 0), (0, OP - O)))
    b2p = jnp.pad(b2, (0, OP - O))

    degp = _make_deg(N, E)(dst, edge_weights)
    g1 = _mm_scale(x, W1, degp)
    p1 = _make_agg(N, H, E)(g1, src, dst, edge_weights)
    g2 = _mid(p1, g1, degp, b1.reshape(1, H), W2p)
    p2 = _make_agg(N, OP, E)(g2, src, dst, edge_weights)
    return _final(p2, g2, degp, b2p.reshape(1, OP), O)


# speedup vs baseline: 1.0949x; 1.0949x over previous
import functools

import jax
import jax.numpy as jnp
from jax import lax
from jax.experimental import pallas as pl
from jax.experimental.pallas import tpu as pltpu
from jax.experimental.pallas import tpu_sc as plsc

NC, NS, LANES = 2, 16, 16
NW = NC * NS

_MESH = dict(core_axis_name="c", subcore_axis_name="s", num_cores=NC,
             num_subcores=NS)
_SC_PARAMS = pltpu.CompilerParams(needs_layout_passes=False)


_K = 80
_CH = 80


def _make_agg(N, C, E):
    K = _K
    ept = E // NW
    nb = ept // K
    nt = nb // 2
    nch = N // _CH
    tmax = -(-nch // NS)
    mesh = plsc.VectorSubcoreMesh(**_MESH)

    @functools.partial(
        pl.kernel,
        out_type=jax.ShapeDtypeStruct((NC, N, C), jnp.float32),
        mesh=mesh,
        scratch_types=[
            pltpu.VMEM((K,), jnp.int32),
            pltpu.VMEM((K,), jnp.int32),
            pltpu.VMEM((K,), jnp.int32),
            pltpu.VMEM((K,), jnp.int32),
            pltpu.VMEM((K,), jnp.int32),
            pltpu.VMEM((K,), jnp.int32),
            pltpu.VMEM((K,), jnp.float32),
            pltpu.VMEM((K,), jnp.float32),
            pltpu.VMEM((K, C), jnp.float32),
            pltpu.VMEM((K, C), jnp.float32),
            pltpu.VMEM_SHARED((N, C), jnp.float32),
            pltpu.SemaphoreType.DMA,
            pltpu.SemaphoreType.DMA,
            pltpu.SemaphoreType.DMA,
            pltpu.SemaphoreType.DMA,
        ],
        compiler_params=_SC_PARAMS,
    )
    def agg(g_hbm, src_hbm, dst_hbm, w_hbm, out_hbm,
            src_a, src_b, dst_a, dst_b, dsts_a, dsts_b, w_a, w_b,
            rows_a, rows_b, acc_sh,
            sem_ia, sem_ib, sem_ga, sem_gb):
        c = lax.axis_index("c")
        s = lax.axis_index("s")
        wid = c * NS + s
        base = wid * ept

        zv = jnp.zeros((LANES,), jnp.float32)

        def zrow(r, carry):
            for j in range(C // LANES):
                rows_a[r, pl.ds(j * LANES, LANES)] = zv
            return carry

        lax.fori_loop(0, _CH, zrow, 0)
        for t in range(tmax):
            ck = s + NS * t

            @pl.when(ck < nch)
            def _():
                pltpu.sync_copy(rows_a.at[pl.ds(0, _CH)],
                                acc_sh.at[pl.ds(ck * _CH, _CH)])

        plsc.subcore_barrier()

        def idx_descs(i, srcb, dstb, wb, sem):
            off = base + i * K
            return (
                pltpu.make_async_copy(src_hbm.at[pl.ds(off, K)], srcb, sem),
                pltpu.make_async_copy(dst_hbm.at[pl.ds(off, K)], dstb, sem),
                pltpu.make_async_copy(w_hbm.at[pl.ds(off, K)], wb, sem),
            )

        def idx_load(i, srcb, dstb, wb, sem):
            for d in idx_descs(i, srcb, dstb, wb, sem):
                d.start()

        def idx_wait(i, srcb, dstb, wb, sem):
            for d in idx_descs(i, srcb, dstb, wb, sem):
                d.wait()

        def gath(rows, srcb, sem):
            return pltpu.make_async_copy(g_hbm.at[srcb], rows, sem)

        def scale(rows, wb):
            def srow(r, carry):
                wv = plsc.load_gather(wb, [jnp.full((LANES,), r, jnp.int32)])
                for j in range(C // LANES):
                    sl = pl.ds(j * LANES, LANES)
                    rows[r, sl] = rows[r, sl] * wv
                return carry

            lax.fori_loop(0, K, srow, 0)

        def cpidx(dstb, dstsb):
            for j in range(K // LANES):
                sl = pl.ds(j * LANES, LANES)
                dstsb[sl] = dstb[sl]

        idx_load(0, src_a, dst_a, w_a, sem_ia)
        idx_load(1, src_b, dst_b, w_b, sem_ib)

        def step(t, carry):
            i0 = 2 * t
            i1 = i0 + 1

            idx_wait(i0, src_a, dst_a, w_a, sem_ia)
            gath(rows_a, src_a, sem_ga).start()
            idx_wait(i1, src_b, dst_b, w_b, sem_ib)
            gath(rows_b, src_b, sem_gb).start()

            gath(rows_a, src_a, sem_ga).wait()
            scale(rows_a, w_a)
            cpidx(dst_a, dsts_a)
            pltpu.sync_copy(rows_a, acc_sh.at[dsts_a], add=True)

            @pl.when(i0 + 2 < nb)
            def _():
                idx_load(i0 + 2, src_a, dst_a, w_a, sem_ia)

            gath(rows_b, src_b, sem_gb).wait()
            scale(rows_b, w_b)
            cpidx(dst_b, dsts_b)
            pltpu.sync_copy(rows_b, acc_sh.at[dsts_b], add=True)

            @pl.when(i1 + 2 < nb)
            def _():
                idx_load(i1 + 2, src_b, dst_b, w_b, sem_ib)

            return carry

        lax.fori_loop(0, nt, step, 0)

        if nb % 2:
            last = nb - 1
            idx_wait(last, src_a, dst_a, w_a, sem_ia)
            gath(rows_a, src_a, sem_ga).start()
            gath(rows_a, src_a, sem_ga).wait()
            scale(rows_a, w_a)
            cpidx(dst_a, dsts_a)
            pltpu.sync_copy(rows_a, acc_sh.at[dsts_a], add=True)

        plsc.subcore_barrier()
        for t in range(tmax):
            ck = s + NS * t

            @pl.when(ck < nch)
            def _():
                pltpu.sync_copy(acc_sh.at[pl.ds(ck * _CH, _CH)],
                                out_hbm.at[c, pl.ds(ck * _CH, _CH)])

    return agg


def _make_deg(N, E):
    C = LANES
    K = _K
    ept = E // NW
    nb = ept // K
    nt = nb // 2
    nch = N // _CH
    tmax = -(-nch // NS)
    mesh = plsc.VectorSubcoreMesh(**_MESH)

    @functools.partial(
        pl.kernel,
        out_type=jax.ShapeDtypeStruct((NC, N, C), jnp.float32),
        mesh=mesh,
        scratch_types=[
            pltpu.VMEM((K,), jnp.int32),
            pltpu.VMEM((K,), jnp.int32),
            pltpu.VMEM((K,), jnp.int32),
            pltpu.VMEM((K,), jnp.int32),
            pltpu.VMEM((K,), jnp.float32),
            pltpu.VMEM((K,), jnp.float32),
            pltpu.VMEM((K, C), jnp.float32),
            pltpu.VMEM((K, C), jnp.float32),
            pltpu.VMEM_SHARED((N, C), jnp.float32),
            pltpu.SemaphoreType.DMA,
            pltpu.SemaphoreType.DMA,
        ],
        compiler_params=_SC_PARAMS,
    )
    def deg(dst_hbm, w_hbm, out_hbm,
            dst_a, dst_b, dsts_a, dsts_b, w_a, w_b, rows_a, rows_b,
            acc_sh, sem_ia, sem_ib):
        c = lax.axis_index("c")
        s = lax.axis_index("s")
        wid = c * NS + s
        base = wid * ept

        zv = jnp.zeros((LANES,), jnp.float32)

        def zrow(r, carry):
            rows_a[r, pl.ds(0, LANES)] = zv
            return carry

        lax.fori_loop(0, _CH, zrow, 0)
        for t in range(tmax):
            ck = s + NS * t

            @pl.when(ck < nch)
            def _():
                pltpu.sync_copy(rows_a.at[pl.ds(0, _CH)],
                                acc_sh.at[pl.ds(ck * _CH, _CH)])

        plsc.subcore_barrier()

        def idx_descs(i, dstb, wb, sem):
            off = base + i * K
            return (
                pltpu.make_async_copy(dst_hbm.at[pl.ds(off, K)], dstb, sem),
                pltpu.make_async_copy(w_hbm.at[pl.ds(off, K)], wb, sem),
            )

        def idx_load(i, dstb, wb, sem):
            for d in idx_descs(i, dstb, wb, sem):
                d.start()

        def idx_wait(i, dstb, wb, sem):
            for d in idx_descs(i, dstb, wb, sem):
                d.wait()

        def fill(rows, wb):
            def frow(r, carry):
                wv = plsc.load_gather(wb, [jnp.full((LANES,), r, jnp.int32)])
                rows[r, pl.ds(0, LANES)] = wv
                return carry

            lax.fori_loop(0, K, frow, 0)

        def cpidx(dstb, dstsb):
            for j in range(K // LANES):
                sl = pl.ds(j * LANES, LANES)
                dstsb[sl] = dstb[sl]

        idx_load(0, dst_a, w_a, sem_ia)
        idx_load(1, dst_b, w_b, sem_ib)

        def step(t, carry):
            i0 = 2 * t
            i1 = i0 + 1

            idx_wait(i0, dst_a, w_a, sem_ia)
            fill(rows_a, w_a)
            cpidx(dst_a, dsts_a)
            pltpu.sync_copy(rows_a, acc_sh.at[dsts_a], add=True)

            @pl.when(i0 + 2 < nb)
            def _():
                idx_load(i0 + 2, dst_a, w_a, sem_ia)

            idx_wait(i1, dst_b, w_b, sem_ib)
            fill(rows_b, w_b)
            cpidx(dst_b, dsts_b)
            pltpu.sync_copy(rows_b, acc_sh.at[dsts_b], add=True)

            @pl.when(i1 + 2 < nb)
            def _():
                idx_load(i1 + 2, dst_b, w_b, sem_ib)

            return carry

        lax.fori_loop(0, nt, step, 0)

        if nb % 2:
            last = nb - 1
            idx_wait(last, dst_a, w_a, sem_ia)
            fill(rows_a, w_a)
            cpidx(dst_a, dsts_a)
            pltpu.sync_copy(rows_a, acc_sh.at[dsts_a], add=True)

        plsc.subcore_barrier()
        for t in range(tmax):
            ck = s + NS * t

            @pl.when(ck < nch)
            def _():
                pltpu.sync_copy(acc_sh.at[pl.ds(ck * _CH, _CH)],
                                out_hbm.at[c, pl.ds(ck * _CH, _CH)])

    return deg


_BR = 400


def _dis_of(degp_ref):
    deg = degp_ref[0] + degp_ref[1]
    return lax.rsqrt(deg[:, 0:1] + 1.0)


def _mm_scale(x, W, degp):
    N, IC = x.shape
    H = W.shape[1]

    def body(x_ref, w_ref, degp_ref, o_ref):
        dis = _dis_of(degp_ref)
        h = jnp.dot(x_ref[...], w_ref[...], preferred_element_type=jnp.float32)
        o_ref[...] = h * dis

    return pl.pallas_call(
        body,
        grid=(N // _BR,),
        in_specs=[
            pl.BlockSpec((_BR, IC), lambda i: (i, 0)),
            pl.BlockSpec((IC, H), lambda i: (0, 0)),
            pl.BlockSpec((NC, _BR, LANES), lambda i: (0, i, 0)),
        ],
        out_specs=pl.BlockSpec((_BR, H), lambda i: (i, 0)),
        out_shape=jax.ShapeDtypeStruct((N, H), jnp.float32),
    )(x, W, degp)


def _mid(p, g1, degp, b1, W2):
    N, H = g1.shape
    O = W2.shape[1]

    def body(p_ref, g1_ref, degp_ref, b1_ref, w2_ref, o_ref):
        dis = _dis_of(degp_ref)
        t = dis * (p_ref[0] + p_ref[1] + g1_ref[...]) + b1_ref[...]
        h1 = jnp.maximum(t, 0.0)
        h2 = jnp.dot(h1, w2_ref[...], preferred_element_type=jnp.float32)
        o_ref[...] = h2 * dis

    return pl.pallas_call(
        body,
        grid=(N // _BR,),
        in_specs=[
            pl.BlockSpec((NC, _BR, H), lambda i: (0, i, 0)),
            pl.BlockSpec((_BR, H), lambda i: (i, 0)),
            pl.BlockSpec((NC, _BR, LANES), lambda i: (0, i, 0)),
            pl.BlockSpec((1, H), lambda i: (0, 0)),
            pl.BlockSpec((H, O), lambda i: (0, 0)),
        ],
        out_specs=pl.BlockSpec((_BR, O), lambda i: (i, 0)),
        out_shape=jax.ShapeDtypeStruct((N, O), jnp.float32),
    )(p, g1, degp, b1, W2)


def _final(p, g2, degp, b2, O):
    N, OP = g2.shape

    def body(p_ref, g2_ref, degp_ref, b2_ref, o_ref):
        dis = _dis_of(degp_ref)
        o = (dis * (p_ref[0] + p_ref[1] + g2_ref[...]) + b2_ref[...])[:, :O]
        m = jnp.max(o, axis=1, keepdims=True)
        e = jnp.exp(o - m)
        lse = jnp.log(jnp.sum(e, axis=1, keepdims=True))
        o_ref[...] = o - m - lse

    return pl.pallas_call(
        body,
        grid=(N // _BR,),
        in_specs=[
            pl.BlockSpec((NC, _BR, OP), lambda i: (0, i, 0)),
            pl.BlockSpec((_BR, OP), lambda i: (i, 0)),
            pl.BlockSpec((NC, _BR, LANES), lambda i: (0, i, 0)),
            pl.BlockSpec((1, OP), lambda i: (0, 0)),
        ],
        out_specs=pl.BlockSpec((_BR, O), lambda i: (i, 0)),
        out_shape=jax.ShapeDtypeStruct((N, O), jnp.float32),
    )(p, g2, degp, b2)


def kernel(x, adj, edge_weights, W1, b1, W2, b2):
    N, IC = x.shape
    H = W1.shape[1]
    O = W2.shape[1]
    E = adj.shape[1]
    src = adj[0]
    dst = adj[1]

    OP = 128
    W2p = jnp.pad(W2, ((0, 0), (0, OP - O)))
    b2p = jnp.pad(b2, (0, OP - O))

    degp = _make_deg(N, E)(dst, edge_weights)
    g1 = _mm_scale(x, W1, degp)
    p1 = _make_agg(N, H, E)(g1, src, dst, edge_weights)
    g2 = _mid(p1, g1, degp, b1.reshape(1, H), W2p)
    p2 = _make_agg(N, OP, E)(g2, src, dst, edge_weights)
    return _final(p2, g2, degp, b2p.reshape(1, OP), O)
